# baseline (speedup 1.0000x reference)
import jax
import jax.numpy as jnp
from jax.experimental import pallas as pl

_C = 256
_BB = 16


def _onehot_body(x_ref, o_ref):
    x = x_ref[...]
    shifted = jnp.concatenate([x[:, 1:], x[:, -1:] + 1], axis=1)
    shifted = jnp.where(shifted > 255, 0, shifted)
    cls = jax.lax.broadcasted_iota(jnp.int32, (x.shape[0], x.shape[1], _C), 2)
    o_ref[...] = (shifted[:, :, None] == cls).astype(jnp.float32)


def kernel(inputs):
    x = inputs.astype(jnp.int32)
    B, T = x.shape
    return pl.pallas_call(
        _onehot_body,
        grid=(B // _BB,),
        in_specs=[pl.BlockSpec((_BB, T), lambda i: (i, 0))],
        out_specs=pl.BlockSpec((_BB, T, _C), lambda i: (i, 0, 0)),
        out_shape=jax.ShapeDtypeStruct((B, T, _C), jnp.float32),
    )(x)


# speedup vs baseline: 1.1404x; 1.1404x over previous
import jax
import jax.numpy as jnp
from jax.experimental import pallas as pl

_C = 256
_BB = 32


def _onehot_body(x_ref, o_ref):
    x = x_ref[...]
    shifted = jnp.concatenate([x[:, 1:], x[:, -1:] + 1], axis=1)
    shifted = jnp.where(shifted > 255, 0, shifted)
    cls = jax.lax.broadcasted_iota(jnp.int32, (x.shape[0], x.shape[1], _C), 2)
    o_ref[...] = (shifted[:, :, None] == cls).astype(jnp.float32)


def kernel(inputs):
    x = inputs.astype(jnp.int32)
    B, T = x.shape
    return pl.pallas_call(
        _onehot_body,
        grid=(B // _BB,),
        in_specs=[pl.BlockSpec((_BB, T), lambda i: (i, 0))],
        out_specs=pl.BlockSpec((_BB, T, _C), lambda i: (i, 0, 0)),
        out_shape=jax.ShapeDtypeStruct((B, T, _C), jnp.float32),
    )(x)
